# baseline (speedup 1.0000x reference)
import functools

import jax
import jax.numpy as jnp
from jax import lax
from jax.experimental import pallas as pl
from jax.experimental.pallas import tpu as pltpu
from jax.experimental.pallas import tpu_sc as plsc

NUM_CHARS = 1000
BATCH = 4096
HIST = 50
NC = 2
NS = 16
NW = NC * NS
BPW = BATCH // NW
NBUF = 2
HP = 56
DP = 1024


@functools.partial(
    pl.kernel,
    mesh=plsc.VectorSubcoreMesh(core_axis_name="c", subcore_axis_name="s"),
    out_type=jax.ShapeDtypeStruct((BATCH, HP, DP), jnp.float32),
    scratch_types=(
        [pltpu.VMEM((BPW * HP,), jnp.int32)]
        + [pltpu.VMEM((HP, DP), jnp.float32) for _ in range(NBUF)]
        + [pltpu.SemaphoreType.DMA for _ in range(2 * NBUF)]
    ),
    compiler_params=pltpu.CompilerParams(use_tc_tiling_on_sc=False),
)
def _emb_gather(idx_hbm, table_hbm, out_hbm, idx_v, *bufs_and_sems):
    rows = bufs_and_sems[:NBUF]
    gsem = bufs_and_sems[NBUF:2 * NBUF]
    ssem = bufs_and_sems[2 * NBUF:]

    wid = lax.axis_index("s") * NC + lax.axis_index("c")
    base = wid * BPW
    pltpu.sync_copy(idx_hbm.at[pl.ds(base * HP, BPW * HP)], idx_v)

    def chunk(b, g, drain_scatter):
        dst = out_hbm.at[base + g]
        if drain_scatter:
            pltpu.make_async_copy(rows[b], dst, ssem[b]).wait()
        pltpu.async_copy(
            table_hbm.at[idx_v.at[pl.ds(g * HP, HP)]], rows[b], gsem[b]
        ).wait()
        pltpu.async_copy(rows[b], dst, ssem[b])

    for b in range(NBUF):
        chunk(b, b, drain_scatter=False)

    def group(go, carry):
        for b in range(NBUF):
            chunk(b, go * NBUF + b, drain_scatter=True)
        return carry

    lax.fori_loop(1, BPW // NBUF, group, 0)

    for b in range(NBUF):
        g = BPW - NBUF + b
        pltpu.make_async_copy(rows[b], out_hbm.at[base + g], ssem[b]).wait()


def kernel(indices, emb_weight):
    idx_pad = jnp.pad(indices.astype(jnp.int32), ((0, 0), (0, HP - HIST)))
    tbl_pad = jnp.pad(emb_weight, ((0, 0), (0, DP - NUM_CHARS)))
    out = _emb_gather(idx_pad.reshape(-1), tbl_pad)
    return out[:, :HIST, :NUM_CHARS]


# speedup vs baseline: 1.3035x; 1.3035x over previous
import functools

import jax
import jax.numpy as jnp
from jax import lax
from jax.experimental import pallas as pl
from jax.experimental.pallas import tpu as pltpu
from jax.experimental.pallas import tpu_sc as plsc

NUM_CHARS = 1000
BATCH = 4096
HIST = 50
NC = 2
NS = 16
NW = NC * NS
BPW = BATCH // NW
NBUF = 2
HP = 56
DP = 1024


@functools.partial(
    pl.kernel,
    mesh=plsc.VectorSubcoreMesh(core_axis_name="c", subcore_axis_name="s"),
    out_type=jax.ShapeDtypeStruct((BATCH, HP, DP), jnp.float32),
    scratch_types=(
        [pltpu.VMEM((HP,), jnp.int32) for _ in range(NBUF)]
        + [pltpu.VMEM((HP, DP + 128), jnp.float32) for _ in range(NBUF)]
        + [pltpu.SemaphoreType.DMA for _ in range(2 * NBUF)]
    ),
)
def _emb_gather(idx_hbm, table_hbm, out_hbm, *bufs_and_sems):
    idxs = bufs_and_sems[:NBUF]
    rows = bufs_and_sems[NBUF:2 * NBUF]
    gsem = bufs_and_sems[2 * NBUF:3 * NBUF]
    ssem = bufs_and_sems[3 * NBUF:]

    wid = lax.axis_index("s") * NC + lax.axis_index("c")
    base = wid * BPW

    def chunk(b, g, drain_scatter):
        dst = out_hbm.at[base + g]
        src = rows[b].at[:, pl.ds(0, DP)]
        if drain_scatter:
            pltpu.make_async_copy(src, dst, ssem[b]).wait()
        pltpu.sync_copy(idx_hbm.at[pl.ds((base + g) * HP, HP)], idxs[b])
        pltpu.async_copy(table_hbm.at[idxs[b]], src, gsem[b]).wait()
        pltpu.async_copy(src, dst, ssem[b])

    for b in range(NBUF):
        chunk(b, b, drain_scatter=False)

    def group(go, carry):
        for b in range(NBUF):
            chunk(b, go * NBUF + b, drain_scatter=True)
        return carry

    lax.fori_loop(1, BPW // NBUF, group, 0)

    for b in range(NBUF):
        g = BPW - NBUF + b
        pltpu.make_async_copy(
            rows[b].at[:, pl.ds(0, DP)], out_hbm.at[base + g], ssem[b]
        ).wait()


def kernel(indices, emb_weight):
    idx_pad = jnp.pad(indices.astype(jnp.int32), ((0, 0), (0, HP - HIST)))
    tbl_pad = jnp.pad(emb_weight, ((0, 0), (0, DP - NUM_CHARS)))
    out = _emb_gather(idx_pad.reshape(-1), tbl_pad)
    return out[:, :HIST, :NUM_CHARS]


# speedup vs baseline: 1.5212x; 1.1670x over previous
import functools

import jax
import jax.numpy as jnp
from jax import lax
from jax.experimental import pallas as pl
from jax.experimental.pallas import tpu as pltpu
from jax.experimental.pallas import tpu_sc as plsc

NUM_CHARS = 1000
BATCH = 4096
HIST = 50
NC = 2
NS = 16
NW = NC * NS
BPW = BATCH // NW
NBUF = 2
HP = 56
DP = 1152


@functools.partial(
    pl.kernel,
    mesh=plsc.VectorSubcoreMesh(core_axis_name="c", subcore_axis_name="s"),
    out_type=jax.ShapeDtypeStruct((BATCH, HIST, DP), jnp.float32),
    scratch_types=(
        [pltpu.VMEM((BPW * HP,), jnp.int32)]
        + [pltpu.VMEM((HIST, DP), jnp.float32) for _ in range(NBUF)]
        + [pltpu.SemaphoreType.DMA for _ in range(2 * NBUF)]
    ),
    compiler_params=pltpu.CompilerParams(use_tc_tiling_on_sc=False),
)
def _emb_gather(idx_hbm, table_hbm, out_hbm, idx_v, *bufs_and_sems):
    rows = bufs_and_sems[:NBUF]
    gsem = bufs_and_sems[NBUF:2 * NBUF]
    ssem = bufs_and_sems[2 * NBUF:]

    wid = lax.axis_index("s") * NC + lax.axis_index("c")
    base = wid * BPW
    pltpu.sync_copy(idx_hbm.at[pl.ds(base * HP, BPW * HP)], idx_v)

    def chunk(b, g, drain_scatter):
        dst = out_hbm.at[base + g]
        if drain_scatter:
            pltpu.make_async_copy(rows[b], dst, ssem[b]).wait()
        pltpu.async_copy(
            table_hbm.at[idx_v.at[pl.ds(g * HP, HIST)]], rows[b], gsem[b]
        ).wait()
        pltpu.async_copy(rows[b], dst, ssem[b])

    for b in range(NBUF):
        chunk(b, b, drain_scatter=False)

    def group(go, carry):
        for b in range(NBUF):
            chunk(b, go * NBUF + b, drain_scatter=True)
        return carry

    lax.fori_loop(1, BPW // NBUF, group, 0)

    for b in range(NBUF):
        g = BPW - NBUF + b
        pltpu.make_async_copy(rows[b], out_hbm.at[base + g], ssem[b]).wait()


def kernel(indices, emb_weight):
    idx_pad = jnp.pad(indices.astype(jnp.int32), ((0, 0), (0, HP - HIST)))
    tbl_pad = jnp.pad(emb_weight, ((0, 0), (0, DP - NUM_CHARS)))
    out = _emb_gather(idx_pad.reshape(-1), tbl_pad)
    return out[:, :, :NUM_CHARS]


# speedup vs baseline: 1.8583x; 1.2216x over previous
import functools

import jax
import jax.numpy as jnp
from jax import lax
from jax.experimental import pallas as pl
from jax.experimental.pallas import tpu as pltpu
from jax.experimental.pallas import tpu_sc as plsc

NUM_CHARS = 1000
BATCH = 4096
HIST = 50
NC = 2
NS = 16
NW = NC * NS
BPW = BATCH // NW
NBUF = 2
HP = 56
DP = 1152
DOUT = 1024


@functools.partial(
    pl.kernel,
    mesh=plsc.VectorSubcoreMesh(core_axis_name="c", subcore_axis_name="s"),
    out_type=jax.ShapeDtypeStruct((BATCH, HIST, DOUT), jnp.float32),
    scratch_types=(
        [pltpu.VMEM((BPW * HP,), jnp.int32)]
        + [pltpu.VMEM((HIST, DP), jnp.float32) for _ in range(NBUF)]
        + [pltpu.SemaphoreType.DMA for _ in range(2 * NBUF)]
    ),
    compiler_params=pltpu.CompilerParams(use_tc_tiling_on_sc=False),
)
def _emb_gather(idx_hbm, table_hbm, out_hbm, idx_v, *bufs_and_sems):
    rows = bufs_and_sems[:NBUF]
    gsem = bufs_and_sems[NBUF:2 * NBUF]
    ssem = bufs_and_sems[2 * NBUF:]

    wid = lax.axis_index("s") * NC + lax.axis_index("c")
    base = wid * BPW
    pltpu.sync_copy(idx_hbm.at[pl.ds(base * HP, BPW * HP)], idx_v)

    def chunk(b, g, drain_scatter):
        dst = out_hbm.at[base + g]
        src = rows[b].at[:, pl.ds(0, DOUT)]
        if drain_scatter:
            pltpu.make_async_copy(src, dst, ssem[b]).wait()
        pltpu.async_copy(
            table_hbm.at[idx_v.at[pl.ds(g * HP, HIST)]], rows[b], gsem[b]
        ).wait()
        pltpu.async_copy(src, dst, ssem[b])

    for b in range(NBUF):
        chunk(b, b, drain_scatter=False)

    def group(go, carry):
        for b in range(NBUF):
            chunk(b, go * NBUF + b, drain_scatter=True)
        return carry

    lax.fori_loop(1, BPW // NBUF, group, 0)

    for b in range(NBUF):
        g = BPW - NBUF + b
        pltpu.make_async_copy(
            rows[b].at[:, pl.ds(0, DOUT)], out_hbm.at[base + g], ssem[b]
        ).wait()


def kernel(indices, emb_weight):
    idx_pad = jnp.pad(indices.astype(jnp.int32), ((0, 0), (0, HP - HIST)))
    tbl_pad = jnp.pad(emb_weight, ((0, 0), (0, DP - NUM_CHARS)))
    out = _emb_gather(idx_pad.reshape(-1), tbl_pad)
    return out[:, :, :NUM_CHARS]


# speedup vs baseline: 1.8704x; 1.0066x over previous
import functools

import jax
import jax.numpy as jnp
from jax import lax
from jax.experimental import pallas as pl
from jax.experimental.pallas import tpu as pltpu
from jax.experimental.pallas import tpu_sc as plsc

NUM_CHARS = 1000
BATCH = 4096
HIST = 50
NC = 2
NS = 16
NW = NC * NS
BPW = BATCH // NW
NBUF = 2
HP = 56
DP = 1152
DOUT = 1024


@functools.partial(
    pl.kernel,
    mesh=plsc.VectorSubcoreMesh(core_axis_name="c", subcore_axis_name="s"),
    out_type=jax.ShapeDtypeStruct((BATCH, HIST, DOUT), jnp.float32),
    scratch_types=(
        [pltpu.VMEM((BPW * HP,), jnp.int32)]
        + [pltpu.VMEM((HIST, DP), jnp.float32) for _ in range(NBUF)]
        + [pltpu.SemaphoreType.DMA for _ in range(2 * NBUF)]
    ),
    compiler_params=pltpu.CompilerParams(use_tc_tiling_on_sc=False),
)
def _emb_gather(idx_hbm, table_hbm, out_hbm, idx_v, *bufs_and_sems):
    rows = bufs_and_sems[:NBUF]
    gsem = bufs_and_sems[NBUF:2 * NBUF]
    ssem = bufs_and_sems[2 * NBUF:]

    wid = lax.axis_index("s") * NC + lax.axis_index("c")
    base = wid * BPW
    pltpu.sync_copy(idx_hbm.at[pl.ds(base * HP, BPW * HP)], idx_v)

    def gather(b, g):
        return pltpu.make_async_copy(
            table_hbm.at[idx_v.at[pl.ds(g * HP, HIST)]], rows[b], gsem[b]
        )

    def scatter(b, g):
        return pltpu.make_async_copy(
            rows[b].at[:, pl.ds(0, DOUT)], out_hbm.at[base + g], ssem[b]
        )

    for b in range(NBUF):
        gather(b, b).start()

    def group(go, carry):
        for b in range(NBUF):
            g = go * NBUF + b
            gather(b, g).wait()
            scatter(b, g).start()
        for b in range(NBUF):
            g = go * NBUF + b
            scatter(b, g).wait()
            gather(b, g + NBUF).start()
        return carry

    lax.fori_loop(0, BPW // NBUF - 1, group, 0)

    for b in range(NBUF):
        g = BPW - NBUF + b
        gather(b, g).wait()
        scatter(b, g).start()
    for b in range(NBUF):
        scatter(b, BPW - NBUF + b).wait()


def kernel(indices, emb_weight):
    idx_pad = jnp.pad(indices.astype(jnp.int32), ((0, 0), (0, HP - HIST)))
    tbl_pad = jnp.pad(emb_weight, ((0, 0), (0, DP - NUM_CHARS)))
    out = _emb_gather(idx_pad.reshape(-1), tbl_pad)
    return out[:, :, :NUM_CHARS]
